# baseline (speedup 1.0000x reference)
import functools

import jax
import jax.numpy as jnp
from jax.experimental import pallas as pl
from jax.experimental.pallas import tpu as pltpu

K = 16
BR = 256
S = 128
_BIGI = 1 << 30


def _sq_kernel(xn_ref, ones_ref, sqc_ref):
    xsq = xn_ref[...] * xn_ref[...]
    sqc_ref[...] = jax.lax.dot_general(
        ones_ref[...], xsq, (((1,), (1,)), ((), ())),
        precision=jax.lax.Precision.HIGHEST,
        preferred_element_type=jnp.float32)


def _dist_block(xr, xc, sqc):
    inner = jax.lax.dot_general(
        xr, xc, (((1,), (1,)), ((), ())),
        precision=jax.lax.Precision.DEFAULT,
        preferred_element_type=jnp.float32)
    return (-2.0 * inner) + sqc[0:1, :]


def _select16_t(dt, colt):
    outs = []
    for _ in range(K):
        m = jnp.min(dt, axis=0, keepdims=True)
        csel = jnp.min(jnp.where(dt == m, colt, _BIGI), axis=0, keepdims=True)
        outs.append(csel)
        dt = jnp.where(colt == csel, jnp.inf, dt)
    return jnp.concatenate(outs, axis=0)


def _knn_kernel2(npad, xr_ref, xc_ref, sqc_ref, out_ref):
    r = npad // S
    dist = _dist_block(xr_ref[...], xc_ref[...], sqc_ref[...])
    lane = jax.lax.broadcasted_iota(jnp.int32, (BR, S), 1)
    f = dist[:, :S]
    cmin = lane
    for a in range(1, r):
        sl = dist[:, a * S:(a + 1) * S]
        cmin = jnp.where(sl < f, a * S + lane, cmin)
        f = jnp.minimum(f, sl)
    fw = f.T
    cmt = cmin.T
    subl = jax.lax.broadcasted_iota(jnp.int32, (S, BR), 0)
    lanes_sel = []
    for _ in range(K):
        m = jnp.min(fw, axis=0, keepdims=True)
        csel = jnp.min(jnp.where(fw == m, cmt, _BIGI), axis=0, keepdims=True)
        lsel = csel & (S - 1)
        lanes_sel.append(lsel)
        fw = jnp.where(subl == lsel, jnp.inf, fw)
    lsel16 = jnp.concatenate(lanes_sel, axis=0).T
    dpieces = []
    cpieces = []
    for a in range(r):
        dpieces.append(jnp.take_along_axis(dist[:, a * S:(a + 1) * S],
                                           lsel16, axis=1))
        cpieces.append(a * S + lsel16)
    cand = jnp.concatenate(dpieces, axis=1)
    candc = jnp.concatenate(cpieces, axis=1)
    w = r * K
    if w % S == 0 and w // S >= 2:
        r2 = w // S
        f2 = cand[:, :S]
        c2 = candc[:, :S]
        for a in range(1, r2):
            sl = cand[:, a * S:(a + 1) * S]
            slc = candc[:, a * S:(a + 1) * S]
            c2 = jnp.where(sl < f2, slc, c2)
            f2 = jnp.minimum(f2, sl)
        fw2 = f2.T
        cmt2 = c2.T
        subl2 = jax.lax.broadcasted_iota(jnp.int32, (S, BR), 0)
        lanes2 = []
        for _ in range(K):
            m = jnp.min(fw2, axis=0, keepdims=True)
            csel = jnp.min(jnp.where(fw2 == m, cmt2, _BIGI), axis=0,
                           keepdims=True)
            lsel = jnp.min(jnp.where(cmt2 == csel, subl2, _BIGI), axis=0,
                           keepdims=True)
            lanes2.append(lsel)
            fw2 = jnp.where(subl2 == lsel, jnp.inf, fw2)
        lsel16b = jnp.concatenate(lanes2, axis=0).T
        d2p = []
        c2p = []
        for a in range(r2):
            d2p.append(jnp.take_along_axis(cand[:, a * S:(a + 1) * S],
                                           lsel16b, axis=1))
            c2p.append(jnp.take_along_axis(candc[:, a * S:(a + 1) * S],
                                           lsel16b, axis=1))
        cand = jnp.concatenate(d2p, axis=1)
        candc = jnp.concatenate(c2p, axis=1)
    out_ref[...] = _select16_t(cand.T, candc.T).T


def _knn_kernel_flat(npad, xr_ref, xc_ref, sqc_ref, out_ref):
    dist = _dist_block(xr_ref[...], xc_ref[...], sqc_ref[...])
    col = jax.lax.broadcasted_iota(jnp.int32, (BR, npad), 1)
    out_ref[...] = _select16_t(dist.T, col.T).T


def kernel(x):
    n, d = x.shape
    npad = ((n + BR - 1) // BR) * BR

    norm = jnp.linalg.norm(x, ord=2, axis=0, keepdims=True)
    xn = x / jnp.maximum(norm, 1e-12)
    xn = jnp.pad(xn, ((0, npad - n), (0, 0)), constant_values=1e4)

    sqc = pl.pallas_call(
        _sq_kernel,
        out_shape=jax.ShapeDtypeStruct((8, npad), jnp.float32),
    )(xn, jnp.ones((8, d), jnp.float32))

    if npad % S == 0 and npad // S >= 2:
        body = functools.partial(_knn_kernel2, npad)
    else:
        body = functools.partial(_knn_kernel_flat, npad)
    grid = npad // BR
    nn = pl.pallas_call(
        body,
        grid=(grid,),
        in_specs=[
            pl.BlockSpec((BR, d), lambda i: (i, 0)),
            pl.BlockSpec((npad, d), lambda i: (0, 0)),
            pl.BlockSpec((8, npad), lambda i: (0, 0)),
        ],
        out_specs=pl.BlockSpec((BR, K), lambda i: (i, 0)),
        out_shape=jax.ShapeDtypeStruct((npad, K), jnp.int32),
    )(xn, xn, sqc)

    nn_idx = nn[:n]
    center_idx = jnp.broadcast_to(jnp.arange(n, dtype=nn_idx.dtype)[:, None],
                                  (n, K))
    return jnp.stack((nn_idx, center_idx), axis=0)


# speedup vs baseline: 1.3736x; 1.3736x over previous
import functools

import jax
import jax.numpy as jnp
from jax.experimental import pallas as pl
from jax.experimental.pallas import tpu as pltpu

K = 16
BR = 256
S = 128
_BIGI = 1 << 30


def _sq_kernel(xn_ref, ones_ref, sqc_ref):
    xsq = xn_ref[...] * xn_ref[...]
    sqc_ref[...] = jax.lax.dot_general(
        ones_ref[...], xsq, (((1,), (1,)), ((), ())),
        precision=jax.lax.Precision.HIGHEST,
        preferred_element_type=jnp.float32)


def _dist_block(xr, xc, sqc):
    inner = jax.lax.dot_general(
        xr, xc, (((1,), (1,)), ((), ())),
        precision=jax.lax.Precision.DEFAULT,
        preferred_element_type=jnp.float32)
    return (-2.0 * inner) + sqc[0:1, :]


def _select16_t(dt, colt):
    outs = []
    for _ in range(K):
        m = jnp.min(dt, axis=0, keepdims=True)
        csel = jnp.min(jnp.where(dt == m, colt, _BIGI), axis=0, keepdims=True)
        outs.append(csel)
        dt = jnp.where(colt == csel, jnp.inf, dt)
    return jnp.concatenate(outs, axis=0)


def _knn_kernel2(npad, xr_ref, xc_ref, sqc_ref, out_ref):
    r = npad // S
    dist = _dist_block(xr_ref[...], xc_ref[...], sqc_ref[...])
    lane = jax.lax.broadcasted_iota(jnp.int32, (BR, S), 1)
    f = dist[:, :S]
    cmin = lane
    for a in range(1, r):
        sl = dist[:, a * S:(a + 1) * S]
        cmin = jnp.where(sl < f, a * S + lane, cmin)
        f = jnp.minimum(f, sl)
    fw = f.T
    cmt = cmin.T
    subl = jax.lax.broadcasted_iota(jnp.int32, (S, BR), 0)
    lanes_sel = []
    for _ in range(K):
        m = jnp.min(fw, axis=0, keepdims=True)
        csel = jnp.min(jnp.where(fw == m, cmt, _BIGI), axis=0, keepdims=True)
        lsel = csel & (S - 1)
        lanes_sel.append(lsel)
        fw = jnp.where(subl == lsel, jnp.inf, fw)
    lsel16 = jnp.concatenate(lanes_sel, axis=0).T
    g_total = r // 8
    sub = lane >> 4
    jrep = jnp.take_along_axis(lsel16, lane & (K - 1), axis=1)
    colbase = (sub << 7) + jrep
    merged = []
    for g in range(g_total):
        mg = jnp.take_along_axis(dist[:, (8 * g) * S:(8 * g + 1) * S],
                                 jrep, axis=1)
        for j in range(1, 8):
            pj = jnp.take_along_axis(
                dist[:, (8 * g + j) * S:(8 * g + j + 1) * S], jrep, axis=1)
            mg = jnp.where(sub == j, pj, mg)
        merged.append(mg)
    if g_total <= 2:
        cand = jnp.concatenate(merged, axis=1)
        candc = jnp.concatenate(
            [g * (8 * S) + colbase for g in range(g_total)], axis=1)
        out_ref[...] = _select16_t(cand.T, candc.T).T
        return
    f2 = merged[0]
    c2 = colbase
    for g in range(1, g_total):
        c2 = jnp.where(merged[g] < f2, g * (8 * S) + colbase, c2)
        f2 = jnp.minimum(f2, merged[g])
    fw2 = f2.T
    cmt2 = c2.T
    lanes2 = []
    for _ in range(K):
        m = jnp.min(fw2, axis=0, keepdims=True)
        csel = jnp.min(jnp.where(fw2 == m, cmt2, _BIGI), axis=0,
                       keepdims=True)
        lsel = jnp.min(jnp.where(cmt2 == csel, subl, _BIGI), axis=0,
                       keepdims=True)
        lanes2.append(lsel)
        fw2 = jnp.where(subl == lsel, jnp.inf, fw2)
    lsel16b = jnp.concatenate(lanes2, axis=0).T
    w2 = g_total * K
    lane2 = jax.lax.broadcasted_iota(jnp.int32, (BR, w2), 1)
    jrep2 = jnp.take_along_axis(lsel16b, lane2 & (K - 1), axis=1)
    cand2 = jnp.take_along_axis(merged[0], jrep2, axis=1)
    for g in range(1, g_total):
        pg = jnp.take_along_axis(merged[g], jrep2, axis=1)
        cand2 = jnp.where((lane2 >> 4) == g, pg, cand2)
    candc2 = ((lane2 >> 4) * (8 * S)
              + jnp.take_along_axis(colbase, jrep2, axis=1))
    out_ref[...] = _select16_t(cand2.T, candc2.T).T


def _knn_kernel_flat(npad, xr_ref, xc_ref, sqc_ref, out_ref):
    dist = _dist_block(xr_ref[...], xc_ref[...], sqc_ref[...])
    col = jax.lax.broadcasted_iota(jnp.int32, (BR, npad), 1)
    out_ref[...] = _select16_t(dist.T, col.T).T


def kernel(x):
    n, d = x.shape
    npad = ((n + BR - 1) // BR) * BR

    norm = jnp.linalg.norm(x, ord=2, axis=0, keepdims=True)
    xn = x / jnp.maximum(norm, 1e-12)
    xn = jnp.pad(xn, ((0, npad - n), (0, 0)), constant_values=1e4)

    sqc = pl.pallas_call(
        _sq_kernel,
        out_shape=jax.ShapeDtypeStruct((8, npad), jnp.float32),
    )(xn, jnp.ones((8, d), jnp.float32))

    if npad % (8 * S) == 0:
        body = functools.partial(_knn_kernel2, npad)
    else:
        body = functools.partial(_knn_kernel_flat, npad)
    grid = npad // BR
    nn = pl.pallas_call(
        body,
        grid=(grid,),
        in_specs=[
            pl.BlockSpec((BR, d), lambda i: (i, 0)),
            pl.BlockSpec((npad, d), lambda i: (0, 0)),
            pl.BlockSpec((8, npad), lambda i: (0, 0)),
        ],
        out_specs=pl.BlockSpec((BR, K), lambda i: (i, 0)),
        out_shape=jax.ShapeDtypeStruct((npad, K), jnp.int32),
    )(xn, xn, sqc)

    nn_idx = nn[:n]
    center_idx = jnp.broadcast_to(jnp.arange(n, dtype=nn_idx.dtype)[:, None],
                                  (n, K))
    return jnp.stack((nn_idx, center_idx), axis=0)


# speedup vs baseline: 1.3974x; 1.0174x over previous
import functools

import jax
import jax.numpy as jnp
from jax.experimental import pallas as pl
from jax.experimental.pallas import tpu as pltpu

K = 16
BR = 512
S = 128
_BIGI = 1 << 30


def _sq_kernel(xn_ref, ones_ref, sqc_ref):
    xsq = xn_ref[...] * xn_ref[...]
    sqc_ref[...] = jax.lax.dot_general(
        ones_ref[...], xsq, (((1,), (1,)), ((), ())),
        precision=jax.lax.Precision.HIGHEST,
        preferred_element_type=jnp.float32)


def _dist_block(xr, xc, sqc):
    inner = jax.lax.dot_general(
        xr, xc, (((1,), (1,)), ((), ())),
        precision=jax.lax.Precision.DEFAULT,
        preferred_element_type=jnp.float32)
    return (-2.0 * inner) + sqc[0:1, :]


def _select16_t(dt, colt):
    outs = []
    for _ in range(K):
        m = jnp.min(dt, axis=0, keepdims=True)
        csel = jnp.min(jnp.where(dt == m, colt, _BIGI), axis=0, keepdims=True)
        outs.append(csel)
        dt = jnp.where(colt == csel, jnp.inf, dt)
    return jnp.concatenate(outs, axis=0)


def _knn_kernel2(npad, xr_ref, xc_ref, sqc_ref, out_ref):
    r = npad // S
    dist = _dist_block(xr_ref[...], xc_ref[...], sqc_ref[...])
    lane = jax.lax.broadcasted_iota(jnp.int32, (BR, S), 1)
    f = dist[:, :S]
    cmin = lane
    for a in range(1, r):
        sl = dist[:, a * S:(a + 1) * S]
        cmin = jnp.where(sl < f, a * S + lane, cmin)
        f = jnp.minimum(f, sl)
    fw = f.T
    cmt = cmin.T
    subl = jax.lax.broadcasted_iota(jnp.int32, (S, BR), 0)
    lanes_sel = []
    for _ in range(K):
        m = jnp.min(fw, axis=0, keepdims=True)
        csel = jnp.min(jnp.where(fw == m, cmt, _BIGI), axis=0, keepdims=True)
        lsel = csel & (S - 1)
        lanes_sel.append(lsel)
        fw = jnp.where(subl == lsel, jnp.inf, fw)
    lsel16 = jnp.concatenate(lanes_sel, axis=0).T
    g_total = r // 8
    sub = lane >> 4
    jrep = jnp.take_along_axis(lsel16, lane & (K - 1), axis=1)
    colbase = (sub << 7) + jrep
    merged = []
    for g in range(g_total):
        mg = jnp.take_along_axis(dist[:, (8 * g) * S:(8 * g + 1) * S],
                                 jrep, axis=1)
        for j in range(1, 8):
            pj = jnp.take_along_axis(
                dist[:, (8 * g + j) * S:(8 * g + j + 1) * S], jrep, axis=1)
            mg = jnp.where(sub == j, pj, mg)
        merged.append(mg)
    if g_total <= 2:
        cand = jnp.concatenate(merged, axis=1)
        candc = jnp.concatenate(
            [g * (8 * S) + colbase for g in range(g_total)], axis=1)
        out_ref[...] = _select16_t(cand.T, candc.T).T
        return
    f2 = merged[0]
    c2 = colbase
    for g in range(1, g_total):
        c2 = jnp.where(merged[g] < f2, g * (8 * S) + colbase, c2)
        f2 = jnp.minimum(f2, merged[g])
    fw2 = f2.T
    cmt2 = c2.T
    lanes2 = []
    for _ in range(K):
        m = jnp.min(fw2, axis=0, keepdims=True)
        csel = jnp.min(jnp.where(fw2 == m, cmt2, _BIGI), axis=0,
                       keepdims=True)
        lsel = jnp.min(jnp.where(cmt2 == csel, subl, _BIGI), axis=0,
                       keepdims=True)
        lanes2.append(lsel)
        fw2 = jnp.where(subl == lsel, jnp.inf, fw2)
    lsel16b = jnp.concatenate(lanes2, axis=0).T
    w2 = g_total * K
    lane2 = jax.lax.broadcasted_iota(jnp.int32, (BR, w2), 1)
    jrep2 = jnp.take_along_axis(lsel16b, lane2 & (K - 1), axis=1)
    cand2 = jnp.take_along_axis(merged[0], jrep2, axis=1)
    for g in range(1, g_total):
        pg = jnp.take_along_axis(merged[g], jrep2, axis=1)
        cand2 = jnp.where((lane2 >> 4) == g, pg, cand2)
    candc2 = ((lane2 >> 4) * (8 * S)
              + jnp.take_along_axis(colbase, jrep2, axis=1))
    out_ref[...] = _select16_t(cand2.T, candc2.T).T


def _knn_kernel_flat(npad, xr_ref, xc_ref, sqc_ref, out_ref):
    dist = _dist_block(xr_ref[...], xc_ref[...], sqc_ref[...])
    col = jax.lax.broadcasted_iota(jnp.int32, (BR, npad), 1)
    out_ref[...] = _select16_t(dist.T, col.T).T


def kernel(x):
    n, d = x.shape
    npad = ((n + BR - 1) // BR) * BR

    norm = jnp.linalg.norm(x, ord=2, axis=0, keepdims=True)
    xn = x / jnp.maximum(norm, 1e-12)
    xn = jnp.pad(xn, ((0, npad - n), (0, 0)), constant_values=1e4)

    sqc = pl.pallas_call(
        _sq_kernel,
        out_shape=jax.ShapeDtypeStruct((8, npad), jnp.float32),
    )(xn, jnp.ones((8, d), jnp.float32))

    if npad % (8 * S) == 0:
        body = functools.partial(_knn_kernel2, npad)
    else:
        body = functools.partial(_knn_kernel_flat, npad)
    grid = npad // BR
    nn = pl.pallas_call(
        body,
        grid=(grid,),
        in_specs=[
            pl.BlockSpec((BR, d), lambda i: (i, 0)),
            pl.BlockSpec((npad, d), lambda i: (0, 0)),
            pl.BlockSpec((8, npad), lambda i: (0, 0)),
        ],
        out_specs=pl.BlockSpec((BR, K), lambda i: (i, 0)),
        out_shape=jax.ShapeDtypeStruct((npad, K), jnp.int32),
    )(xn, xn, sqc)

    nn_idx = nn[:n]
    center_idx = jnp.broadcast_to(jnp.arange(n, dtype=nn_idx.dtype)[:, None],
                                  (n, K))
    return jnp.stack((nn_idx, center_idx), axis=0)
